# Initial kernel scaffold; baseline (speedup 1.0000x reference)
import jax
import jax.numpy as jnp
from jax.experimental import pallas as pl


def kernel(x, edge_index, W1, b1, W2, b2):
    raise NotImplementedError("write your pallas kernel here")


# speedup vs baseline: 14.6700x; 14.6700x over previous
import functools

import jax
import jax.numpy as jnp
from jax import lax
from jax.experimental import pallas as pl
from jax.experimental.pallas import tpu as pltpu
from jax.experimental.pallas import tpu_sc as plsc

N = 10000
E = 320000
D = 128

NC = 2
NS = 16
NW = NC * NS

CH = 128
NCHUNK = E // CH
BASE_CH = NCHUNK // NW
EXTRA = NCHUNK - BASE_CH * NW

NP = 10240
ROWS_PT = NP // NS
DEG_PT = NP // NS

_mesh = plsc.VectorSubcoreMesh(
    core_axis_name="c", subcore_axis_name="s", num_cores=NC, num_subcores=NS)


def _tile_chunks(wid):
    start = BASE_CH * wid + jnp.minimum(wid, EXTRA)
    count = BASE_CH + (wid < EXTRA).astype(jnp.int32)
    return start, count


def _deg_body(dst_hbm, d0_hbm, d1_hbm, deg_sp, zv, ones_v, didx, sem):
    cid = lax.axis_index("c")
    sid = lax.axis_index("s")
    wid = cid * NS + sid

    def zinit(i, carry):
        zv[pl.ds(i * 16, 16)] = jnp.zeros((16,), jnp.float32)
        return carry

    lax.fori_loop(0, DEG_PT // 16, zinit, None)

    def oinit(i, carry):
        ones_v[pl.ds(i * 16, 16)] = jnp.ones((16,), jnp.float32)
        return carry

    lax.fori_loop(0, CH // 16, oinit, None)

    sl = pl.ds(sid * DEG_PT, DEG_PT)
    pltpu.sync_copy(zv, deg_sp.at[sl])
    plsc.subcore_barrier()

    start, count = _tile_chunks(wid)

    def body(c, carry):
        b = pl.multiple_of((start + c) * CH, CH)
        pltpu.sync_copy(dst_hbm.at[pl.ds(b, CH)], didx)
        pltpu.sync_copy(ones_v, deg_sp.at[didx], add=True)
        return carry

    lax.fori_loop(0, count, body, None)
    plsc.subcore_barrier()

    @pl.when(cid == 0)
    def _():
        pltpu.sync_copy(deg_sp.at[sl], d0_hbm.at[sl])

    @pl.when(cid == 1)
    def _():
        pltpu.sync_copy(deg_sp.at[sl], d1_hbm.at[sl])


_deg_kernel = pl.kernel(
    _deg_body,
    out_type=(jax.ShapeDtypeStruct((NP,), jnp.float32),
              jax.ShapeDtypeStruct((NP,), jnp.float32)),
    mesh=_mesh,
    scratch_types=[
        pltpu.VMEM_SHARED((NP,), jnp.float32),
        pltpu.VMEM((DEG_PT,), jnp.float32),
        pltpu.VMEM((CH,), jnp.float32),
        pltpu.VMEM((CH,), jnp.int32),
        pltpu.SemaphoreType.DMA,
    ],
)


def _scat_body(src_hbm, dst_hbm, y_hbm, z_hbm, o0_hbm, o1_hbm,
               acc_sp, sidx, didx, rows, sem):
    cid = lax.axis_index("c")
    sid = lax.axis_index("s")
    wid = cid * NS + sid

    rsl = pl.ds(sid * ROWS_PT, ROWS_PT)
    pltpu.sync_copy(z_hbm.at[rsl], acc_sp.at[rsl])
    plsc.subcore_barrier()

    start, count = _tile_chunks(wid)

    def body(c, carry):
        b = pl.multiple_of((start + c) * CH, CH)
        pltpu.sync_copy(src_hbm.at[pl.ds(b, CH)], sidx)
        pltpu.sync_copy(dst_hbm.at[pl.ds(b, CH)], didx)
        pltpu.async_copy(y_hbm.at[sidx], rows, sem).wait()
        pltpu.sync_copy(rows, acc_sp.at[didx], add=True)
        return carry

    lax.fori_loop(0, count, body, None)
    plsc.subcore_barrier()

    @pl.when(cid == 0)
    def _():
        pltpu.sync_copy(acc_sp.at[rsl], o0_hbm.at[rsl])

    @pl.when(cid == 1)
    def _():
        pltpu.sync_copy(acc_sp.at[rsl], o1_hbm.at[rsl])


_scat_kernel = pl.kernel(
    _scat_body,
    out_type=(jax.ShapeDtypeStruct((NP, D), jnp.float32),
              jax.ShapeDtypeStruct((NP, D), jnp.float32)),
    mesh=_mesh,
    scratch_types=[
        pltpu.VMEM_SHARED((NP, D), jnp.float32),
        pltpu.VMEM((CH,), jnp.int32),
        pltpu.VMEM((CH,), jnp.int32),
        pltpu.VMEM((CH, D), jnp.float32),
        pltpu.SemaphoreType.DMA,
    ],
)

R = 200
G = N // R


def _tc1_body(x_ref, w_ref, d0_ref, d1_ref, dinv_ref, y1_ref):
    dv = lax.rsqrt(d0_ref[...] + d1_ref[...] + 1.0)
    dinv_ref[...] = dv
    y1_ref[...] = jnp.dot(
        x_ref[...], w_ref[...], preferred_element_type=jnp.float32) * dv


def _tc2_body(s0_ref, s1_ref, y1_ref, dv_ref, b1_ref, w2_ref, y2_ref):
    dv = dv_ref[...]
    h = jnp.maximum(dv * (s0_ref[...] + s1_ref[...] + y1_ref[...]) + b1_ref[...],
                    0.0)
    y2_ref[...] = jnp.dot(
        h, w2_ref[...], preferred_element_type=jnp.float32) * dv


def _tc3_body(s0_ref, s1_ref, y2_ref, dv_ref, b2_ref, out_ref):
    out_ref[...] = dv_ref[...] * (s0_ref[...] + s1_ref[...] + y2_ref[...]) \
        + b2_ref[...]


_row_spec = pl.BlockSpec((R, D), lambda i: (i, 0))
_col_spec = pl.BlockSpec((R, 1), lambda i: (i, 0))
_w_spec = pl.BlockSpec((D, D), lambda i: (0, 0))
_b_spec = pl.BlockSpec((1, D), lambda i: (0, 0))

_tc1 = pl.pallas_call(
    _tc1_body,
    grid=(G,),
    in_specs=[_row_spec, _w_spec, _col_spec, _col_spec],
    out_specs=[_col_spec, _row_spec],
    out_shape=(jax.ShapeDtypeStruct((N, 1), jnp.float32),
               jax.ShapeDtypeStruct((N, D), jnp.float32)),
)

_tc2 = pl.pallas_call(
    _tc2_body,
    grid=(G,),
    in_specs=[_row_spec, _row_spec, _row_spec, _col_spec, _b_spec, _w_spec],
    out_specs=_row_spec,
    out_shape=jax.ShapeDtypeStruct((N, D), jnp.float32),
)

_tc3 = pl.pallas_call(
    _tc3_body,
    grid=(G,),
    in_specs=[_row_spec, _row_spec, _row_spec, _col_spec, _b_spec],
    out_specs=_row_spec,
    out_shape=jax.ShapeDtypeStruct((N, D), jnp.float32),
)


def kernel(x, edge_index, W1, b1, W2, b2):
    src = edge_index[0]
    dst = edge_index[1]
    zeros = jnp.zeros((NP, D), jnp.float32)

    d0, d1 = _deg_kernel(dst)
    d0 = d0[:N].reshape(N, 1)
    d1 = d1[:N].reshape(N, 1)

    dinv, y1 = _tc1(x, W1, d0, d1)
    s10, s11 = _scat_kernel(src, dst, y1, zeros)
    y2 = _tc2(s10, s11, y1, dinv, b1.reshape(1, D), W2)
    s20, s21 = _scat_kernel(src, dst, y2, zeros)
    out = _tc3(s20, s21, y2, dinv, b2.reshape(1, D))
    return out
